# baseline (speedup 1.0000x reference)
import functools

import jax
import jax.numpy as jnp
from jax import lax
from jax.experimental import pallas as pl
from jax.experimental.pallas import tpu as pltpu
from jax.experimental.pallas import tpu_sc as plsc

_B = 16384
_D = 64
_NC = 2
_NS = 16
_NW = _NC * _NS
_BPW = _B // _NW
_L = 16
_CR = 32
_NCHK = _BPW // _CR
_NBLK = 500000
_W = 2 * _D

_mesh = plsc.VectorSubcoreMesh(core_axis_name="c", subcore_axis_name="s")


@functools.partial(
    pl.kernel,
    mesh=_mesh,
    compiler_params=pltpu.CompilerParams(needs_layout_passes=False),
    out_type=jax.ShapeDtypeStruct((_B,), jnp.float32),
    scratch_types=[
        pltpu.VMEM((_BPW,), jnp.int32),
        pltpu.VMEM((_BPW,), jnp.int32),
        pltpu.VMEM((_NCHK, _CR), jnp.int32),
        pltpu.VMEM((_NCHK, _CR), jnp.int32),
        pltpu.VMEM((2, _CR, _W), jnp.float32),
        pltpu.VMEM((2, _CR, _W), jnp.float32),
        pltpu.VMEM((_BPW,), jnp.float32),
        pltpu.VMEM((_L * _L,), jnp.float32),
        pltpu.SemaphoreType.DMA,
        pltpu.SemaphoreType.DMA,
    ],
)
def _sc_dot(bidx_hbm, uidx_hbm, ut_hbm, bt_hbm, out_hbm,
            bidx_v, uidx_v, bblk_v, ublk_v, bbuf_v, ubuf_v, out_v, tbuf_v,
            sem0, sem1):
    cid = lax.axis_index("c")
    sid = lax.axis_index("s")
    wid = sid * _NC + cid
    base = wid * _BPW

    pltpu.sync_copy(bidx_hbm.at[pl.ds(base, _BPW)], bidx_v)
    pltpu.sync_copy(uidx_hbm.at[pl.ds(base, _BPW)], uidx_v)

    lane = lax.iota(jnp.int32, _L)
    lane16 = lane * _L

    def blk_body(i, carry):
        ch = i // (_CR // _L)
        off = (i % (_CR // _L)) * _L
        bblk_v[ch, pl.ds(off, _L)] = bidx_v[pl.ds(i * _L, _L)] >> 1
        ublk_v[ch, pl.ds(off, _L)] = uidx_v[pl.ds(i * _L, _L)] >> 1
        return carry

    lax.fori_loop(0, _BPW // _L, blk_body, 0)

    def enqueue(ch, p, sem):
        pltpu.async_copy(bt_hbm.at[bblk_v.at[ch]], bbuf_v.at[p], sem)
        pltpu.async_copy(ut_hbm.at[ublk_v.at[ch]], ubuf_v.at[p], sem)

    def wait(ch, p, sem):
        pltpu.make_async_copy(bt_hbm.at[bblk_v.at[ch]], bbuf_v.at[p], sem).wait()
        pltpu.make_async_copy(ut_hbm.at[ublk_v.at[ch]], ubuf_v.at[p], sem).wait()

    def compute(ch, p):
        for half in range(_CR // _L):
            r0 = ch * _CR + half * _L
            offs_u = (uidx_v[pl.ds(r0, _L)] & 1) * _D
            offs_b = (bidx_v[pl.ds(r0, _L)] & 1) * _D
            for k in range(_L):
                j = half * _L + k
                su = offs_u[k]
                sb = offs_b[k]
                t0 = (ubuf_v[p, j, pl.ds(su, _L)]
                      * bbuf_v[p, j, pl.ds(sb, _L)])
                t1 = (ubuf_v[p, j, pl.ds(su + _L, _L)]
                      * bbuf_v[p, j, pl.ds(sb + _L, _L)])
                t2 = (ubuf_v[p, j, pl.ds(su + 2 * _L, _L)]
                      * bbuf_v[p, j, pl.ds(sb + 2 * _L, _L)])
                t3 = (ubuf_v[p, j, pl.ds(su + 3 * _L, _L)]
                      * bbuf_v[p, j, pl.ds(sb + 3 * _L, _L)])
                t = (t0 + t1) + (t2 + t3)
                plsc.store_scatter(tbuf_v, [lane16 + k], t)
            cols = [tbuf_v[pl.ds(l * _L, _L)] for l in range(_L)]
            while len(cols) > 1:
                cols = [cols[i] + cols[i + 1] for i in range(0, len(cols), 2)]
            out_v[pl.ds(r0, _L)] = cols[0]

    enqueue(0, 0, sem0)

    def pipe_body(g, carry):
        i0 = 2 * g
        enqueue(i0 + 1, 1, sem1)
        wait(i0, 0, sem0)
        compute(i0, 0)

        @pl.when(g < _NCHK // 2 - 1)
        def _():
            enqueue(i0 + 2, 0, sem0)

        wait(i0 + 1, 1, sem1)
        compute(i0 + 1, 1)
        return carry

    lax.fori_loop(0, _NCHK // 2, pipe_body, 0)

    pltpu.sync_copy(out_v, out_hbm.at[pl.ds(base, _BPW)])


def kernel(inputs, user_table, book_table):
    book_idx = inputs[:, 0]
    user_idx = inputs[:, 1]
    ut2 = user_table.reshape(_NBLK, _W)
    bt2 = book_table.reshape(_NBLK, _W)
    out = _sc_dot(book_idx, user_idx, ut2, bt2)
    return out.reshape(_B, 1)


# speedup vs baseline: 2.2094x; 2.2094x over previous
import functools

import jax
import jax.numpy as jnp
from jax import lax
from jax.experimental import pallas as pl
from jax.experimental.pallas import tpu as pltpu
from jax.experimental.pallas import tpu_sc as plsc

_B = 16384
_D = 64
_NC = 2
_NS = 16
_NW = _NC * _NS
_BPW = _B // _NW
_L = 16
_CR = 16
_NCHK = _BPW // _CR
_NBLK = 125000

_mesh = plsc.VectorSubcoreMesh(core_axis_name="c", subcore_axis_name="s")


@functools.partial(
    pl.kernel,
    mesh=_mesh,
    compiler_params=pltpu.CompilerParams(needs_layout_passes=False),
    out_type=jax.ShapeDtypeStruct((_B,), jnp.float32),
    scratch_types=[
        pltpu.VMEM((_BPW,), jnp.int32),
        pltpu.VMEM((_BPW,), jnp.int32),
        pltpu.VMEM((2, _CR, 8, _D), jnp.float32),
        pltpu.VMEM((2, _CR, 8, _D), jnp.float32),
        pltpu.VMEM((_BPW,), jnp.float32),
        pltpu.VMEM((_L * _L,), jnp.float32),
        pltpu.SemaphoreType.DMA,
        pltpu.SemaphoreType.DMA,
    ],
)
def _sc_dot(bidx_hbm, uidx_hbm, ut_hbm, bt_hbm, out_hbm,
            bidx_v, uidx_v, bbuf_v, ubuf_v, out_v, tbuf_v,
            sem0, sem1):
    cid = lax.axis_index("c")
    sid = lax.axis_index("s")
    wid = sid * _NC + cid
    base = wid * _BPW

    pltpu.sync_copy(bidx_hbm.at[pl.ds(base, _BPW)], bidx_v)
    pltpu.sync_copy(uidx_hbm.at[pl.ds(base, _BPW)], uidx_v)

    lane = lax.iota(jnp.int32, _L)
    lane16 = lane * _L

    def enqueue(ch, p, sem):
        for q in range(_CR // _L):
            bblk = bidx_v[pl.ds(ch * _CR + q * _L, _L)] >> 3
            ublk = uidx_v[pl.ds(ch * _CR + q * _L, _L)] >> 3
            for k in range(_L):
                j = q * _L + k
                pltpu.async_copy(bt_hbm.at[bblk[k]], bbuf_v.at[p, j], sem)
                pltpu.async_copy(ut_hbm.at[ublk[k]], ubuf_v.at[p, j], sem)

    def wait(ch, p, sem):
        pltpu.make_async_copy(bt_hbm.at[pl.ds(0, _CR)], bbuf_v.at[p], sem).wait()
        pltpu.make_async_copy(ut_hbm.at[pl.ds(0, _CR)], ubuf_v.at[p], sem).wait()

    def compute(ch, p):
        for half in range(_CR // _L):
            r0 = ch * _CR + half * _L
            subs_u = uidx_v[pl.ds(r0, _L)] & 7
            subs_b = bidx_v[pl.ds(r0, _L)] & 7
            for k in range(_L):
                j = half * _L + k
                su = subs_u[k]
                sb = subs_b[k]
                t0 = (ubuf_v[p, j, su, pl.ds(0, _L)]
                      * bbuf_v[p, j, sb, pl.ds(0, _L)])
                t1 = (ubuf_v[p, j, su, pl.ds(_L, _L)]
                      * bbuf_v[p, j, sb, pl.ds(_L, _L)])
                t2 = (ubuf_v[p, j, su, pl.ds(2 * _L, _L)]
                      * bbuf_v[p, j, sb, pl.ds(2 * _L, _L)])
                t3 = (ubuf_v[p, j, su, pl.ds(3 * _L, _L)]
                      * bbuf_v[p, j, sb, pl.ds(3 * _L, _L)])
                t = (t0 + t1) + (t2 + t3)
                plsc.store_scatter(tbuf_v, [lane16 + k], t)
            cols = [tbuf_v[pl.ds(l * _L, _L)] for l in range(_L)]
            while len(cols) > 1:
                cols = [cols[i] + cols[i + 1] for i in range(0, len(cols), 2)]
            out_v[pl.ds(r0, _L)] = cols[0]

    enqueue(0, 0, sem0)

    def pipe_body(g, carry):
        i0 = 2 * g
        enqueue(i0 + 1, 1, sem1)
        wait(i0, 0, sem0)
        compute(i0, 0)

        @pl.when(g < _NCHK // 2 - 1)
        def _():
            enqueue(i0 + 2, 0, sem0)

        wait(i0 + 1, 1, sem1)
        compute(i0 + 1, 1)
        return carry

    lax.fori_loop(0, _NCHK // 2, pipe_body, 0)

    pltpu.sync_copy(out_v, out_hbm.at[pl.ds(base, _BPW)])


def kernel(inputs, user_table, book_table):
    book_idx = inputs[:, 0]
    user_idx = inputs[:, 1]
    ut3 = user_table.reshape(_NBLK, 8, _D)
    bt3 = book_table.reshape(_NBLK, 8, _D)
    out = _sc_dot(book_idx, user_idx, ut3, bt3)
    return out.reshape(_B, 1)
